# baseline (speedup 1.0000x reference)
import functools

import jax
import jax.numpy as jnp
from jax import lax
from jax.experimental import pallas as pl
from jax.experimental.pallas import tpu as pltpu
from jax.experimental.pallas import tpu_sc as plsc

N = 10000
E = 160000
M = 2000
P = 50000

NC = 2
NS = 16
NW = NC * NS
L = 16

NP = 10240
MP = 2048
E32 = 163840
EWC = E32 // NW

BM = 1000


def _mesh():
    return plsc.VectorSubcoreMesh(core_axis_name="c", subcore_axis_name="s",
                                  num_cores=NC, num_subcores=NS)


def _seg_sorted(tables, ss, dd, bounds, *, op, fc):
    nchunks = len(tables)
    opt = op // NW
    nfc = fc // L

    def body(*refs):
        ss_hbm, dd_hbm, b_hbm = refs[0], refs[1], refs[2]
        tabs = refs[3:3 + nchunks]
        outs = refs[3 + nchunks:3 + 2 * nchunks]
        srci, dsti, rowbuf, staging, bref = refs[3 + 2 * nchunks:]
        cid = lax.axis_index("c")
        sid = lax.axis_index("s")
        w = cid * NS + sid
        lo = w * opt

        pltpu.sync_copy(b_hbm, bref)

        def _sload(idx):
            total = jnp.int32(0)
            for cc in range(3):
                v = bref[pl.ds(cc * L, L)]
                gi = lax.iota(jnp.int32, L) + cc * L
                total = total + jnp.sum(jnp.where(gi == idx, v, 0))
            return total

        b_lo = _sload(w)
        b_hi = _sload(w + 1)
        e0 = (b_lo // 128) * 128
        nbt = (b_hi + 127) // 128 - b_lo // 128

        zero = jnp.zeros((L,), jnp.float32)

        def flush(dp, acc):
            @pl.when((dp >= lo) & (dp < lo + opt))
            def _():
                dl = dp - lo
                for t in range(nfc):
                    staging[dl, pl.ds(t * L, L)] = (
                        staging[dl, pl.ds(t * L, L)] + acc[t])

        for c in range(nchunks):
            def zloop(i, _):
                for t in range(nfc):
                    staging[i, pl.ds(t * L, L)] = zero
                return 0
            lax.fori_loop(0, opt, zloop, 0)

            def bloop(b, carry):
                base = e0 + b * 128
                pltpu.sync_copy(ss_hbm.at[pl.ds(base, 128)], srci)
                pltpu.sync_copy(dd_hbm.at[pl.ds(base, 128)], dsti)
                pltpu.sync_copy(tabs[c].at[srci], rowbuf)

                def gloop(i, gcarry):
                    d16 = dsti[pl.ds(i * L, L)]
                    cur = gcarry
                    for j2 in range(L):
                        dp = cur[0]
                        acc = cur[1:]
                        d = d16[j2]
                        changed = d != dp

                        @pl.when(changed)
                        def _(dp=dp, acc=acc):
                            flush(dp, acc)

                        j = i * L + j2
                        row = [rowbuf[j, pl.ds(t * L, L)] for t in range(nfc)]
                        cur = (d,) + tuple(
                            jnp.where(changed, row[t], acc[t] + row[t])
                            for t in range(nfc))
                    return cur
                return lax.fori_loop(0, 128 // L, gloop, carry)

            init = (jnp.int32(-1),) + tuple(zero for _ in range(nfc))
            fin = lax.fori_loop(0, nbt, bloop, init)
            flush(fin[0], fin[1:])
            pltpu.sync_copy(staging, outs[c].at[pl.ds(lo, opt)])

    out_type = tuple(jax.ShapeDtypeStruct((op, fc), jnp.float32)
                     for _ in range(nchunks))
    fn = pl.kernel(
        body,
        out_type=out_type,
        mesh=_mesh(),
        compiler_params=pltpu.CompilerParams(needs_layout_passes=False),
        scratch_types=[
            pltpu.VMEM((128,), jnp.int32),
            pltpu.VMEM((128,), jnp.int32),
            pltpu.VMEM((128, fc), jnp.float32),
            pltpu.VMEM((opt, fc), jnp.float32),
            pltpu.VMEM((48,), jnp.int32),
        ],
    )
    return fn(ss, dd, bounds, *tables)


def _counts(dst2, node2, he2):
    def body(dst_hbm, node_hbm, he_hbm, od, on, oh, dv, nv, hv, hd, hn, hh):
        cid = lax.axis_index("c")
        sid = lax.axis_index("s")
        w = cid * NS + sid
        pltpu.sync_copy(dst_hbm.at[pl.ds(w * EWC, EWC)], dv)
        pltpu.sync_copy(node_hbm.at[pl.ds(w * EWC, EWC)], nv)
        pltpu.sync_copy(he_hbm.at[pl.ds(w * EWC, EWC)], hv)

        zero = jnp.zeros((L,), jnp.float32)

        def z1(i, _):
            hd[pl.ds(i * L, L)] = zero
            hn[pl.ds(i * L, L)] = zero
            return 0
        lax.fori_loop(0, NP // L, z1, 0)

        def z2(i, _):
            hh[pl.ds(i * L, L)] = zero
            return 0
        lax.fori_loop(0, MP // L, z2, 0)

        ones = jnp.full((L,), 1.0, jnp.float32)

        def eloop(i, _):
            plsc.addupdate_scatter(hd, [dv[pl.ds(i * L, L)]], ones)
            plsc.addupdate_scatter(hn, [nv[pl.ds(i * L, L)]], ones)
            plsc.addupdate_scatter(hh, [hv[pl.ds(i * L, L)]], ones)
            return 0
        lax.fori_loop(0, EWC // L, eloop, 0)

        pltpu.sync_copy(hd, od.at[pl.ds(w * NP, NP)])
        pltpu.sync_copy(hn, on.at[pl.ds(w * NP, NP)])
        pltpu.sync_copy(hh, oh.at[pl.ds(w * MP, MP)])

    fn = pl.kernel(
        body,
        out_type=(jax.ShapeDtypeStruct((NW * NP,), jnp.float32),
                  jax.ShapeDtypeStruct((NW * NP,), jnp.float32),
                  jax.ShapeDtypeStruct((NW * MP,), jnp.float32)),
        mesh=_mesh(),
        compiler_params=pltpu.CompilerParams(needs_layout_passes=False),
        scratch_types=[
            pltpu.VMEM((EWC,), jnp.int32),
            pltpu.VMEM((EWC,), jnp.int32),
            pltpu.VMEM((EWC,), jnp.int32),
            pltpu.VMEM((NP,), jnp.float32),
            pltpu.VMEM((NP,), jnp.float32),
            pltpu.VMEM((MP,), jnp.float32),
        ],
    )
    cd, cn, ch = fn(dst2, node2, he2)
    return (cd.reshape(NW, NP), cn.reshape(NW, NP), ch.reshape(NW, MP))


def _decoder(zw, z, i03, i13, *, ppad, nb, k, fz):
    pt = ppad // NW

    def body(zw_hbm, z_hbm, i0_hbm, i1_hbm, out, i0r, i1r, r0, r1, sv):
        cid = lax.axis_index("c")
        sid = lax.axis_index("s")
        w = cid * NS + sid

        lane = lax.iota(jnp.int32, L)

        def bloop(b, _):
            base = (w * nb + b) * k
            pltpu.sync_copy(i0_hbm.at[pl.ds(base, k)], i0r)
            pltpu.sync_copy(i1_hbm.at[pl.ds(base, k)], i1r)
            pltpu.sync_copy(zw_hbm.at[i0r], r0)
            pltpu.sync_copy(z_hbm.at[i1r], r1)

            def gloop(g, _):
                vec = jnp.zeros((L,), jnp.float32)
                for j in range(L):
                    p = g * L + j
                    acc = r0[p, pl.ds(0, L)] * r1[p, pl.ds(0, L)]
                    for f in range(1, fz // L):
                        acc = acc + r0[p, pl.ds(f * L, L)] * r1[p, pl.ds(f * L, L)]
                    vec = jnp.where(lane == j, jnp.sum(acc, axis=0), vec)
                sv[pl.ds(b * k + g * L, L)] = vec
                return 0
            lax.fori_loop(0, k // L, gloop, 0)
            return 0
        lax.fori_loop(0, nb, bloop, 0)
        pltpu.sync_copy(sv, out.at[pl.ds(w * pt, pt)])

    fn = pl.kernel(
        body,
        out_type=jax.ShapeDtypeStruct((ppad,), jnp.float32),
        mesh=_mesh(),
        compiler_params=pltpu.CompilerParams(needs_layout_passes=False),
        scratch_types=[
            pltpu.VMEM((k,), jnp.int32),
            pltpu.VMEM((k,), jnp.int32),
            pltpu.VMEM((k, fz), jnp.float32),
            pltpu.VMEM((k, fz), jnp.float32),
            pltpu.VMEM((pt,), jnp.float32),
        ],
    )
    return fn(zw, z, i03, i13)


def _elu(v):
    return jnp.where(v > 0, v, jnp.exp(jnp.minimum(v, 0.0)) - 1.0)


def _inv_body(cd_ref, cn_ref, ch_ref, di_ref, dn_ref, bi_ref):
    d = jnp.sum(cd_ref[...], axis=0) + 1.0
    di_ref[...] = lax.rsqrt(d)[:, None]
    cn = jnp.sum(cn_ref[...], axis=0)
    dn_ref[...] = jnp.where(cn > 0, 1.0 / jnp.where(cn > 0, cn, 1.0), 0.0)[:, None]
    ch = jnp.sum(ch_ref[...], axis=0)
    bi_ref[...] = jnp.where(ch > 0, 1.0 / jnp.where(ch > 0, ch, 1.0), 0.0)[:, None]


def _invs(cd, cn, ch):
    return pl.pallas_call(
        _inv_body,
        out_shape=(jax.ShapeDtypeStruct((NP, 1), jnp.float32),
                   jax.ShapeDtypeStruct((NP, 1), jnp.float32),
                   jax.ShapeDtypeStruct((MP, 1), jnp.float32)),
    )(cd, cn, ch)


def _mm_body(a_ref, w_ref, b_ref, o_ref, *, act):
    acc = jnp.dot(a_ref[...], w_ref[...], preferred_element_type=jnp.float32)
    acc = acc + b_ref[...]
    if act == "elu":
        acc = _elu(acc)
    o_ref[...] = acc


def _mm(a, w, bias=None, act=None, bm=BM):
    m, kk = a.shape
    f = w.shape[1]
    if bias is None:
        bias = jnp.zeros((f,), jnp.float32)
    return pl.pallas_call(
        functools.partial(_mm_body, act=act),
        grid=(m // bm,),
        in_specs=[
            pl.BlockSpec((bm, kk), lambda i: (i, 0)),
            pl.BlockSpec((kk, f), lambda i: (0, 0)),
            pl.BlockSpec((f,), lambda i: (0,)),
        ],
        out_specs=pl.BlockSpec((bm, f), lambda i: (i, 0)),
        out_shape=jax.ShapeDtypeStruct((m, f), jnp.float32),
    )(a, w, bias)


def _mm_scale_body(a_ref, w_ref, s_ref, o_ref):
    o_ref[0] = jnp.dot(a_ref[...], w_ref[...],
                       preferred_element_type=jnp.float32) * s_ref[...]


def _mm_scale_chunks(a, w, scale, bm=BM):
    m, kk = a.shape
    f = w.shape[1]
    c = f // 128
    return pl.pallas_call(
        _mm_scale_body,
        grid=(c, m // bm),
        in_specs=[
            pl.BlockSpec((bm, kk), lambda ci, i: (i, 0)),
            pl.BlockSpec((kk, 128), lambda ci, i: (0, ci)),
            pl.BlockSpec((bm, 1), lambda ci, i: (i, 0)),
        ],
        out_specs=pl.BlockSpec((1, bm, 128), lambda ci, i: (ci, i, 0)),
        out_shape=jax.ShapeDtypeStruct((c, m, 128), jnp.float32),
    )(a, w, scale)


def _gcn_mid_body(p0, p1, p2, p3, hn_ref, s_ref, b_ref, w_ref, o_ref):
    parts = [p0, p1, p2, p3]
    t = jnp.concatenate(
        [parts[c][...] + hn_ref[c] for c in range(4)], axis=-1)
    a = _elu(t * s_ref[...] + b_ref[...])
    h2 = jnp.dot(a, w_ref[...], preferred_element_type=jnp.float32) * s_ref[...]
    o_ref[0] = h2[:, :128]
    o_ref[1] = h2[:, 128:]


def _gcn_mid(parts, hn, dinv, b1, w2, bm=BM):
    pspec = pl.BlockSpec((bm, 128), lambda i: (i, 0))
    return pl.pallas_call(
        _gcn_mid_body,
        grid=(N // bm,),
        in_specs=[pspec, pspec, pspec, pspec,
                  pl.BlockSpec((4, bm, 128), lambda i: (0, i, 0)),
                  pl.BlockSpec((bm, 1), lambda i: (i, 0)),
                  pl.BlockSpec((512,), lambda i: (0,)),
                  pl.BlockSpec((512, 256), lambda i: (0, 0))],
        out_specs=pl.BlockSpec((2, bm, 128), lambda i: (0, i, 0)),
        out_shape=jax.ShapeDtypeStruct((2, N, 128), jnp.float32),
    )(*parts, hn, dinv, b1, w2)


def _comb_body(*refs, nch):
    o_ref = refs[-1]
    b_ref = refs[-2]
    s_ref = refs[-3]
    parts = refs[:nch]
    hn_ref = refs[nch] if len(refs) == nch + 4 else None
    cols = []
    for c in range(nch):
        v = parts[c][...]
        if hn_ref is not None:
            v = v + hn_ref[c]
        cols.append(v)
    t = jnp.concatenate(cols, axis=-1)
    o_ref[...] = t * s_ref[...] + b_ref[...]


def _combine(parts, hn, scale, bias, bm=BM):
    nch = len(parts)
    f = nch * 128
    pspec = pl.BlockSpec((bm, 128), lambda i: (i, 0))
    in_specs = [pspec] * nch
    args = list(parts)
    if hn is not None:
        in_specs.append(pl.BlockSpec((nch, bm, 128), lambda i: (0, i, 0)))
        args.append(hn)
    in_specs += [pl.BlockSpec((bm, 1), lambda i: (i, 0)),
                 pl.BlockSpec((f,), lambda i: (0,))]
    args += [scale, bias]
    return pl.pallas_call(
        functools.partial(_comb_body, nch=nch),
        grid=(N // bm,),
        in_specs=in_specs,
        out_specs=pl.BlockSpec((bm, f), lambda i: (i, 0)),
        out_shape=jax.ShapeDtypeStruct((N, f), jnp.float32),
    )(*args)


def _eagg_body(*refs, nq):
    s_ref = refs[nq]
    o_ref = refs[-1]
    for qi in range(nq):
        o_ref[qi] = refs[qi][:M] * s_ref[...][:M]


def _eagg(qs, binv):
    nq = len(qs)
    return pl.pallas_call(
        functools.partial(_eagg_body, nq=nq),
        in_specs=[pl.BlockSpec((MP, 128), lambda: (0, 0))] * nq
        + [pl.BlockSpec((MP, 1), lambda: (0, 0))],
        out_specs=pl.BlockSpec((nq, M, 128), lambda: (0, 0, 0)),
        out_shape=jax.ShapeDtypeStruct((nq, M, 128), jnp.float32),
        grid=(),
    )(*qs, binv)


def _hyp_mid_body(r0, r1, r2, r3, s_ref, b_ref, w_ref, o_ref):
    parts = [r0, r1, r2, r3]
    t = jnp.concatenate([parts[c][...] for c in range(4)], axis=-1)
    a = _elu(t * s_ref[...] + b_ref[...])
    g2 = jnp.dot(a, w_ref[...], preferred_element_type=jnp.float32)
    o_ref[0] = g2[:, :128]
    o_ref[1] = g2[:, 128:]


def _hyp_mid(parts, dinv, bh1, wh2, bm=BM):
    pspec = pl.BlockSpec((bm, 128), lambda i: (i, 0))
    return pl.pallas_call(
        _hyp_mid_body,
        grid=(N // bm,),
        in_specs=[pspec, pspec, pspec, pspec,
                  pl.BlockSpec((bm, 1), lambda i: (i, 0)),
                  pl.BlockSpec((512,), lambda i: (0,)),
                  pl.BlockSpec((512, 256), lambda i: (0, 0))],
        out_specs=pl.BlockSpec((2, bm, 128), lambda i: (0, i, 0)),
        out_shape=jax.ShapeDtypeStruct((2, N, 128), jnp.float32),
    )(*parts, dinv, bh1, wh2)


def _gate_body(xs_ref, xd_ref, g_ref, w_ref, z_ref, zw_ref):
    a = 1.0 / (1.0 + jnp.exp(-g_ref[0, 0]))
    z = a * xs_ref[...] + (1.0 - a) * xd_ref[...]
    z_ref[...] = z
    zw_ref[...] = jnp.dot(z, w_ref[...], preferred_element_type=jnp.float32)


def _gate_fuse(xs, xd, gate, wdec, bm=BM):
    return pl.pallas_call(
        _gate_body,
        grid=(N // bm,),
        in_specs=[pl.BlockSpec((bm, 256), lambda i: (i, 0)),
                  pl.BlockSpec((bm, 256), lambda i: (i, 0)),
                  pl.BlockSpec((1, 1), lambda i: (0, 0)),
                  pl.BlockSpec((256, 256), lambda i: (0, 0))],
        out_specs=(pl.BlockSpec((bm, 256), lambda i: (i, 0)),
                   pl.BlockSpec((bm, 256), lambda i: (i, 0))),
        out_shape=(jax.ShapeDtypeStruct((N, 256), jnp.float32),
                   jax.ShapeDtypeStruct((N, 256), jnp.float32)),
    )(xs, xd, gate, wdec)


def _mlp_body(a_ref, w1_ref, b1_ref, w2_ref, b2_ref, o_ref):
    t = _elu(jnp.dot(a_ref[...], w1_ref[...],
                     preferred_element_type=jnp.float32) + b1_ref[...])
    o_ref[...] = jnp.dot(t, w2_ref[...],
                         preferred_element_type=jnp.float32) + b2_ref[...]


def _mlp(a, w1, b1, w2, b2, bm=BM):
    return pl.pallas_call(
        _mlp_body,
        grid=(N // bm,),
        in_specs=[pl.BlockSpec((bm, 256), lambda i: (i, 0)),
                  pl.BlockSpec((256, 256), lambda i: (0, 0)),
                  pl.BlockSpec((256,), lambda i: (0,)),
                  pl.BlockSpec((256, 256), lambda i: (0, 0)),
                  pl.BlockSpec((256,), lambda i: (0,))],
        out_specs=pl.BlockSpec((bm, 256), lambda i: (i, 0)),
        out_shape=jax.ShapeDtypeStruct((N, 256), jnp.float32),
    )(a, w1, b1, w2, b2)


def _pad_idx(a, pad_val, shape):
    total = 1
    for s in shape:
        total *= s
    a = a.astype(jnp.int32)
    return jnp.concatenate(
        [a, jnp.full((total - a.shape[0],), pad_val, jnp.int32)]).reshape(shape)


def _sort_edges(s_arr, d_arr, pad_d, opt):
    sp = _pad_idx(s_arr, 0, (E32,))
    dp = _pad_idx(d_arr, pad_d, (E32,))
    order = jnp.argsort(dp)
    ss = sp[order]
    dd = dp[order]
    ticks = (jnp.arange(NW + 1) * opt).astype(jnp.int32)
    bounds = jnp.searchsorted(dd, ticks, side="left").astype(jnp.int32)
    bounds = jnp.concatenate([bounds, jnp.zeros((15,), jnp.int32)])
    return ss, dd, bounds


def kernel(x, edge_index, hyperedge_index, pos_edges, neg_edges, W1, b1, W2, b2,
           Wh1, bh1, Wh2, bh2, gate, Wdec, Pw1, Pb1, Pw2, Pb2):
    src, dst = edge_index[0], edge_index[1]
    node, he = hyperedge_index[0], hyperedge_index[1]

    ssg, ddg, bg = _sort_edges(src, dst, NP - 1, NP // NW)
    ss1, dd1, b1h = _sort_edges(node, he, MP - 1, MP // NW)
    ss2, dd2, b2h = _sort_edges(he, node, NP - 1, NP // NW)

    ppad = 102400
    i03 = _pad_idx(jnp.concatenate([pos_edges[0], neg_edges[0]]), 0, (ppad,))
    i13 = _pad_idx(jnp.concatenate([pos_edges[1], neg_edges[1]]), 0, (ppad,))

    cd, cn, ch = _counts(ddg, dd2, dd1)
    dinv_f, ninv_f, binv = _invs(cd, cn, ch)
    dinv = dinv_f[:N]
    ninv = ninv_f[:N]

    hn1 = _mm_scale_chunks(x, W1, dinv)
    p1 = _seg_sorted([hn1[c] for c in range(4)], ssg, ddg, bg, op=NP, fc=128)
    hn2 = _gcn_mid(p1, hn1, dinv, b1, W2)
    p2 = _seg_sorted([hn2[0], hn2[1]], ssg, ddg, bg, op=NP, fc=128)
    x_s = _combine(p2, hn2, dinv, b2)

    ones_n = jnp.ones((N, 1), jnp.float32)
    g1 = _mm_scale_chunks(x, Wh1, ones_n)
    q1 = _seg_sorted([g1[c] for c in range(4)], ss1, dd1, b1h, op=MP, fc=128)
    e1 = _eagg(q1, binv)
    r1 = _seg_sorted([e1[c] for c in range(4)], ss2, dd2, b2h, op=NP, fc=128)
    g2 = _hyp_mid(r1, ninv, bh1, Wh2)
    q2 = _seg_sorted([g2[0], g2[1]], ss1, dd1, b1h, op=MP, fc=128)
    e2 = _eagg(q2, binv)
    r2 = _seg_sorted([e2[0], e2[1]], ss2, dd2, b2h, op=NP, fc=128)
    x_d = _combine(r2, None, ninv, bh2)

    z, zw = _gate_fuse(x_s, x_d, gate.reshape(1, 1), Wdec)
    scores = _decoder(zw, z, i03, i13, ppad=ppad, nb=25, k=128, fz=256)
    pos_scores = scores[:P]
    neg_scores = scores[P:2 * P]

    proj_s = _mlp(x_s, Pw1, Pb1, Pw2, Pb2)
    proj_d = _mlp(x_d, Pw1, Pb1, Pw2, Pb2)
    return (pos_scores, neg_scores, proj_s, proj_d)


# speedup vs baseline: 1.4746x; 1.4746x over previous
import functools

import jax
import jax.numpy as jnp
from jax import lax
from jax.experimental import pallas as pl
from jax.experimental.pallas import tpu as pltpu
from jax.experimental.pallas import tpu_sc as plsc

N = 10000
E = 160000
M = 2000
P = 50000

NC = 2
NS = 16
NW = NC * NS
L = 16

NP = 10240
MP = 2048
E32 = 163840
EWC = E32 // NW

BM = 1000


def _mesh():
    return plsc.VectorSubcoreMesh(core_axis_name="c", subcore_axis_name="s",
                                  num_cores=NC, num_subcores=NS)


def _seg_sorted(tables, ss, rowptr, *, op, fc):
    nchunks = len(tables)
    opt = op // NW
    nfc = fc // L

    def body(*refs):
        ss_hbm, rp_hbm = refs[0], refs[1]
        tabs = refs[2:2 + nchunks]
        outs = refs[2 + nchunks:2 + 2 * nchunks]
        srci, rowbuf, staging, rp_v = refs[2 + 2 * nchunks:]
        cid = lax.axis_index("c")
        sid = lax.axis_index("s")
        w = cid * NS + sid
        lo = w * opt

        pltpu.sync_copy(rp_hbm.at[pl.ds(lo, opt + L)], rp_v)
        b_lo = rp_v[pl.ds(0, L)][0]
        b_hi = rp_v[pl.ds(opt, L)][0]
        e0 = (b_lo // 128) * 128
        nbt = (b_hi + 127) // 128 - b_lo // 128

        zero = jnp.zeros((L,), jnp.float32)

        for c in range(nchunks):
            def zloop(i, _):
                for t in range(nfc):
                    staging[i, pl.ds(t * L, L)] = zero
                return 0
            lax.fori_loop(0, opt, zloop, 0)

            def bloop(b, dcur):
                base = e0 + b * 128
                bend = base + 128
                pltpu.sync_copy(ss_hbm.at[pl.ds(base, 128)], srci)
                pltpu.sync_copy(tabs[c].at[srci], rowbuf)

                def cond(st):
                    d = st[0]
                    v = rp_v[pl.ds(d, L)]
                    return (d < opt) & (v[0] < bend)

                def wbody(st):
                    d = st[0]
                    v = rp_v[pl.ds(d, L)]
                    es = jnp.maximum(v[0] - base, 0)
                    ee = jnp.minimum(v[1] - base, 128)

                    def ebody(e, acc):
                        return tuple(
                            acc[t] + rowbuf[e, pl.ds(t * L, L)]
                            for t in range(nfc))
                    acc = lax.fori_loop(
                        es, ee, ebody, tuple(zero for _ in range(nfc)))
                    for t in range(nfc):
                        staging[d, pl.ds(t * L, L)] = (
                            staging[d, pl.ds(t * L, L)] + acc[t])
                    return (d + 1,)

                d_fin = lax.while_loop(cond, wbody, (dcur,))[0]
                vf = rp_v[pl.ds(d_fin, L)]
                return jnp.where((d_fin > 0) & (vf[0] > bend), d_fin - 1, d_fin)

            lax.fori_loop(0, nbt, bloop, jnp.int32(0))
            pltpu.sync_copy(staging, outs[c].at[pl.ds(lo, opt)])

    out_type = tuple(jax.ShapeDtypeStruct((op, fc), jnp.float32)
                     for _ in range(nchunks))
    fn = pl.kernel(
        body,
        out_type=out_type,
        mesh=_mesh(),
        compiler_params=pltpu.CompilerParams(needs_layout_passes=False),
        scratch_types=[
            pltpu.VMEM((128,), jnp.int32),
            pltpu.VMEM((128, fc), jnp.float32),
            pltpu.VMEM((opt, fc), jnp.float32),
            pltpu.VMEM((opt + L,), jnp.int32),
        ],
    )
    return fn(ss, rowptr, *tables)


def _counts(dst2, node2, he2):
    def body(dst_hbm, node_hbm, he_hbm, od, on, oh, dv, nv, hv, hd, hn, hh):
        cid = lax.axis_index("c")
        sid = lax.axis_index("s")
        w = cid * NS + sid
        pltpu.sync_copy(dst_hbm.at[pl.ds(w * EWC, EWC)], dv)
        pltpu.sync_copy(node_hbm.at[pl.ds(w * EWC, EWC)], nv)
        pltpu.sync_copy(he_hbm.at[pl.ds(w * EWC, EWC)], hv)

        zero = jnp.zeros((L,), jnp.float32)

        def z1(i, _):
            hd[pl.ds(i * L, L)] = zero
            hn[pl.ds(i * L, L)] = zero
            return 0
        lax.fori_loop(0, NP // L, z1, 0)

        def z2(i, _):
            hh[pl.ds(i * L, L)] = zero
            return 0
        lax.fori_loop(0, MP // L, z2, 0)

        ones = jnp.full((L,), 1.0, jnp.float32)

        def eloop(i, _):
            plsc.addupdate_scatter(hd, [dv[pl.ds(i * L, L)]], ones)
            plsc.addupdate_scatter(hn, [nv[pl.ds(i * L, L)]], ones)
            plsc.addupdate_scatter(hh, [hv[pl.ds(i * L, L)]], ones)
            return 0
        lax.fori_loop(0, EWC // L, eloop, 0)

        pltpu.sync_copy(hd, od.at[pl.ds(w * NP, NP)])
        pltpu.sync_copy(hn, on.at[pl.ds(w * NP, NP)])
        pltpu.sync_copy(hh, oh.at[pl.ds(w * MP, MP)])

    fn = pl.kernel(
        body,
        out_type=(jax.ShapeDtypeStruct((NW * NP,), jnp.float32),
                  jax.ShapeDtypeStruct((NW * NP,), jnp.float32),
                  jax.ShapeDtypeStruct((NW * MP,), jnp.float32)),
        mesh=_mesh(),
        compiler_params=pltpu.CompilerParams(needs_layout_passes=False),
        scratch_types=[
            pltpu.VMEM((EWC,), jnp.int32),
            pltpu.VMEM((EWC,), jnp.int32),
            pltpu.VMEM((EWC,), jnp.int32),
            pltpu.VMEM((NP,), jnp.float32),
            pltpu.VMEM((NP,), jnp.float32),
            pltpu.VMEM((MP,), jnp.float32),
        ],
    )
    cd, cn, ch = fn(dst2, node2, he2)
    return (cd.reshape(NW, NP), cn.reshape(NW, NP), ch.reshape(NW, MP))


def _decoder(zw, z, i03, i13, *, ppad, nb, k, fz):
    pt = ppad // NW

    def body(zw_hbm, z_hbm, i0_hbm, i1_hbm, out, i0r, i1r, r0, r1, sv):
        cid = lax.axis_index("c")
        sid = lax.axis_index("s")
        w = cid * NS + sid

        lane = lax.iota(jnp.int32, L)

        def bloop(b, _):
            base = (w * nb + b) * k
            pltpu.sync_copy(i0_hbm.at[pl.ds(base, k)], i0r)
            pltpu.sync_copy(i1_hbm.at[pl.ds(base, k)], i1r)
            pltpu.sync_copy(zw_hbm.at[i0r], r0)
            pltpu.sync_copy(z_hbm.at[i1r], r1)

            def gloop(g, _):
                vec = jnp.zeros((L,), jnp.float32)
                for j in range(L):
                    p = g * L + j
                    acc = r0[p, pl.ds(0, L)] * r1[p, pl.ds(0, L)]
                    for f in range(1, fz // L):
                        acc = acc + r0[p, pl.ds(f * L, L)] * r1[p, pl.ds(f * L, L)]
                    vec = jnp.where(lane == j, jnp.sum(acc, axis=0), vec)
                sv[pl.ds(b * k + g * L, L)] = vec
                return 0
            lax.fori_loop(0, k // L, gloop, 0)
            return 0
        lax.fori_loop(0, nb, bloop, 0)
        pltpu.sync_copy(sv, out.at[pl.ds(w * pt, pt)])

    fn = pl.kernel(
        body,
        out_type=jax.ShapeDtypeStruct((ppad,), jnp.float32),
        mesh=_mesh(),
        compiler_params=pltpu.CompilerParams(needs_layout_passes=False),
        scratch_types=[
            pltpu.VMEM((k,), jnp.int32),
            pltpu.VMEM((k,), jnp.int32),
            pltpu.VMEM((k, fz), jnp.float32),
            pltpu.VMEM((k, fz), jnp.float32),
            pltpu.VMEM((pt,), jnp.float32),
        ],
    )
    return fn(zw, z, i03, i13)


def _elu(v):
    return jnp.where(v > 0, v, jnp.exp(jnp.minimum(v, 0.0)) - 1.0)


def _inv_body(cd_ref, cn_ref, ch_ref, di_ref, dn_ref, bi_ref):
    d = jnp.sum(cd_ref[...], axis=0) + 1.0
    di_ref[...] = lax.rsqrt(d)[:, None]
    cn = jnp.sum(cn_ref[...], axis=0)
    dn_ref[...] = jnp.where(cn > 0, 1.0 / jnp.where(cn > 0, cn, 1.0), 0.0)[:, None]
    ch = jnp.sum(ch_ref[...], axis=0)
    bi_ref[...] = jnp.where(ch > 0, 1.0 / jnp.where(ch > 0, ch, 1.0), 0.0)[:, None]


def _invs(cd, cn, ch):
    return pl.pallas_call(
        _inv_body,
        out_shape=(jax.ShapeDtypeStruct((NP, 1), jnp.float32),
                   jax.ShapeDtypeStruct((NP, 1), jnp.float32),
                   jax.ShapeDtypeStruct((MP, 1), jnp.float32)),
    )(cd, cn, ch)


def _mm_body(a_ref, w_ref, b_ref, o_ref, *, act):
    acc = jnp.dot(a_ref[...], w_ref[...], preferred_element_type=jnp.float32)
    acc = acc + b_ref[...]
    if act == "elu":
        acc = _elu(acc)
    o_ref[...] = acc


def _mm(a, w, bias=None, act=None, bm=BM):
    m, kk = a.shape
    f = w.shape[1]
    if bias is None:
        bias = jnp.zeros((f,), jnp.float32)
    return pl.pallas_call(
        functools.partial(_mm_body, act=act),
        grid=(m // bm,),
        in_specs=[
            pl.BlockSpec((bm, kk), lambda i: (i, 0)),
            pl.BlockSpec((kk, f), lambda i: (0, 0)),
            pl.BlockSpec((f,), lambda i: (0,)),
        ],
        out_specs=pl.BlockSpec((bm, f), lambda i: (i, 0)),
        out_shape=jax.ShapeDtypeStruct((m, f), jnp.float32),
    )(a, w, bias)


def _mm_scale_body(a_ref, w_ref, s_ref, o_ref):
    o_ref[0] = jnp.dot(a_ref[...], w_ref[...],
                       preferred_element_type=jnp.float32) * s_ref[...]


def _mm_scale_chunks(a, w, scale, bm=BM):
    m, kk = a.shape
    f = w.shape[1]
    c = f // 256
    return pl.pallas_call(
        _mm_scale_body,
        grid=(c, m // bm),
        in_specs=[
            pl.BlockSpec((bm, kk), lambda ci, i: (i, 0)),
            pl.BlockSpec((kk, 256), lambda ci, i: (0, ci)),
            pl.BlockSpec((bm, 1), lambda ci, i: (i, 0)),
        ],
        out_specs=pl.BlockSpec((1, bm, 256), lambda ci, i: (ci, i, 0)),
        out_shape=jax.ShapeDtypeStruct((c, m, 256), jnp.float32),
    )(a, w, scale)


def _gcn_mid_body(p0, p1, hn_ref, s_ref, b_ref, w_ref, o_ref):
    t = jnp.concatenate(
        [p0[...] + hn_ref[0], p1[...] + hn_ref[1]], axis=-1)
    a = _elu(t * s_ref[...] + b_ref[...])
    o_ref[...] = jnp.dot(a, w_ref[...],
                         preferred_element_type=jnp.float32) * s_ref[...]


def _gcn_mid(parts, hn, dinv, b1, w2, bm=BM):
    pspec = pl.BlockSpec((bm, 256), lambda i: (i, 0))
    return pl.pallas_call(
        _gcn_mid_body,
        grid=(N // bm,),
        in_specs=[pspec, pspec,
                  pl.BlockSpec((2, bm, 256), lambda i: (0, i, 0)),
                  pl.BlockSpec((bm, 1), lambda i: (i, 0)),
                  pl.BlockSpec((512,), lambda i: (0,)),
                  pl.BlockSpec((512, 256), lambda i: (0, 0))],
        out_specs=pl.BlockSpec((bm, 256), lambda i: (i, 0)),
        out_shape=jax.ShapeDtypeStruct((N, 256), jnp.float32),
    )(*parts, hn, dinv, b1, w2)


def _comb_body(*refs, has_hn):
    o_ref = refs[-1]
    b_ref = refs[-2]
    s_ref = refs[-3]
    t = refs[0][...]
    if has_hn:
        t = t + refs[1][...]
    o_ref[...] = t * s_ref[...] + b_ref[...]


def _combine(part, hn, scale, bias, bm=BM):
    pspec = pl.BlockSpec((bm, 256), lambda i: (i, 0))
    in_specs = [pspec]
    args = [part]
    if hn is not None:
        in_specs.append(pspec)
        args.append(hn)
    in_specs += [pl.BlockSpec((bm, 1), lambda i: (i, 0)),
                 pl.BlockSpec((256,), lambda i: (0,))]
    args += [scale, bias]
    return pl.pallas_call(
        functools.partial(_comb_body, has_hn=hn is not None),
        grid=(N // bm,),
        in_specs=in_specs,
        out_specs=pl.BlockSpec((bm, 256), lambda i: (i, 0)),
        out_shape=jax.ShapeDtypeStruct((N, 256), jnp.float32),
    )(*args)


def _eagg_body(*refs, nq):
    s_ref = refs[nq]
    o_ref = refs[-1]
    for qi in range(nq):
        o_ref[qi] = refs[qi][:M] * s_ref[...][:M]


def _eagg(qs, binv):
    nq = len(qs)
    return pl.pallas_call(
        functools.partial(_eagg_body, nq=nq),
        in_specs=[pl.BlockSpec((MP, 256), lambda: (0, 0))] * nq
        + [pl.BlockSpec((MP, 1), lambda: (0, 0))],
        out_specs=pl.BlockSpec((nq, M, 256), lambda: (0, 0, 0)),
        out_shape=jax.ShapeDtypeStruct((nq, M, 256), jnp.float32),
        grid=(),
    )(*qs, binv)


def _hyp_mid_body(r0, r1, s_ref, b_ref, w_ref, o_ref):
    t = jnp.concatenate([r0[...], r1[...]], axis=-1)
    a = _elu(t * s_ref[...] + b_ref[...])
    o_ref[...] = jnp.dot(a, w_ref[...], preferred_element_type=jnp.float32)


def _hyp_mid(parts, dinv, bh1, wh2, bm=BM):
    pspec = pl.BlockSpec((bm, 256), lambda i: (i, 0))
    return pl.pallas_call(
        _hyp_mid_body,
        grid=(N // bm,),
        in_specs=[pspec, pspec,
                  pl.BlockSpec((bm, 1), lambda i: (i, 0)),
                  pl.BlockSpec((512,), lambda i: (0,)),
                  pl.BlockSpec((512, 256), lambda i: (0, 0))],
        out_specs=pl.BlockSpec((bm, 256), lambda i: (i, 0)),
        out_shape=jax.ShapeDtypeStruct((N, 256), jnp.float32),
    )(*parts, dinv, bh1, wh2)


def _gate_body(xs_ref, xd_ref, g_ref, w_ref, z_ref, zw_ref):
    a = 1.0 / (1.0 + jnp.exp(-g_ref[0, 0]))
    z = a * xs_ref[...] + (1.0 - a) * xd_ref[...]
    z_ref[...] = z
    zw_ref[...] = jnp.dot(z, w_ref[...], preferred_element_type=jnp.float32)


def _gate_fuse(xs, xd, gate, wdec, bm=BM):
    return pl.pallas_call(
        _gate_body,
        grid=(N // bm,),
        in_specs=[pl.BlockSpec((bm, 256), lambda i: (i, 0)),
                  pl.BlockSpec((bm, 256), lambda i: (i, 0)),
                  pl.BlockSpec((1, 1), lambda i: (0, 0)),
                  pl.BlockSpec((256, 256), lambda i: (0, 0))],
        out_specs=(pl.BlockSpec((bm, 256), lambda i: (i, 0)),
                   pl.BlockSpec((bm, 256), lambda i: (i, 0))),
        out_shape=(jax.ShapeDtypeStruct((N, 256), jnp.float32),
                   jax.ShapeDtypeStruct((N, 256), jnp.float32)),
    )(xs, xd, gate, wdec)


def _mlp_body(a_ref, w1_ref, b1_ref, w2_ref, b2_ref, o_ref):
    t = _elu(jnp.dot(a_ref[...], w1_ref[...],
                     preferred_element_type=jnp.float32) + b1_ref[...])
    o_ref[...] = jnp.dot(t, w2_ref[...],
                         preferred_element_type=jnp.float32) + b2_ref[...]


def _mlp(a, w1, b1, w2, b2, bm=BM):
    return pl.pallas_call(
        _mlp_body,
        grid=(N // bm,),
        in_specs=[pl.BlockSpec((bm, 256), lambda i: (i, 0)),
                  pl.BlockSpec((256, 256), lambda i: (0, 0)),
                  pl.BlockSpec((256,), lambda i: (0,)),
                  pl.BlockSpec((256, 256), lambda i: (0, 0)),
                  pl.BlockSpec((256,), lambda i: (0,))],
        out_specs=pl.BlockSpec((bm, 256), lambda i: (i, 0)),
        out_shape=jax.ShapeDtypeStruct((N, 256), jnp.float32),
    )(a, w1, b1, w2, b2)


def _pad_idx(a, pad_val, shape):
    total = 1
    for s in shape:
        total *= s
    a = a.astype(jnp.int32)
    return jnp.concatenate(
        [a, jnp.full((total - a.shape[0],), pad_val, jnp.int32)]).reshape(shape)


def _sort_edges(s_arr, d_arr, pad_d, op):
    sp = _pad_idx(s_arr, 0, (E32,))
    dp = _pad_idx(d_arr, pad_d, (E32,))
    order = jnp.argsort(dp)
    ss = sp[order]
    dd = dp[order]
    ticks = jnp.arange(op + 1, dtype=jnp.int32)
    rowptr = jnp.searchsorted(dd, ticks, side="left").astype(jnp.int32)
    rowptr = jnp.concatenate(
        [rowptr, jnp.full((15,), E32, jnp.int32)])
    return ss, dd, rowptr


def kernel(x, edge_index, hyperedge_index, pos_edges, neg_edges, W1, b1, W2, b2,
           Wh1, bh1, Wh2, bh2, gate, Wdec, Pw1, Pb1, Pw2, Pb2):
    src, dst = edge_index[0], edge_index[1]
    node, he = hyperedge_index[0], hyperedge_index[1]

    ssg, ddg, rpg = _sort_edges(src, dst, NP - 1, NP)
    ss1, dd1, rp1 = _sort_edges(node, he, MP - 1, MP)
    ss2, dd2, rp2 = _sort_edges(he, node, NP - 1, NP)

    ppad = 102400
    i03 = _pad_idx(jnp.concatenate([pos_edges[0], neg_edges[0]]), 0, (ppad,))
    i13 = _pad_idx(jnp.concatenate([pos_edges[1], neg_edges[1]]), 0, (ppad,))

    cd, cn, ch = _counts(ddg, dd2, dd1)
    dinv_f, ninv_f, binv = _invs(cd, cn, ch)
    dinv = dinv_f[:N]
    ninv = ninv_f[:N]

    hn1 = _mm_scale_chunks(x, W1, dinv)
    p1 = _seg_sorted([hn1[0], hn1[1]], ssg, rpg, op=NP, fc=256)
    hn2 = _gcn_mid(p1, hn1, dinv, b1, W2)
    p2 = _seg_sorted([hn2], ssg, rpg, op=NP, fc=256)[0]
    x_s = _combine(p2, hn2, dinv, b2)

    ones_n = jnp.ones((N, 1), jnp.float32)
    g1 = _mm_scale_chunks(x, Wh1, ones_n)
    q1 = _seg_sorted([g1[0], g1[1]], ss1, rp1, op=MP, fc=256)
    e1 = _eagg(q1, binv)
    r1 = _seg_sorted([e1[0], e1[1]], ss2, rp2, op=NP, fc=256)
    g2 = _hyp_mid(r1, ninv, bh1, Wh2)
    q2 = _seg_sorted([g2], ss1, rp1, op=MP, fc=256)
    e2 = _eagg(q2, binv)
    r2 = _seg_sorted([e2[0]], ss2, rp2, op=NP, fc=256)[0]
    x_d = _combine(r2, None, ninv, bh2)

    z, zw = _gate_fuse(x_s, x_d, gate.reshape(1, 1), Wdec)
    scores = _decoder(zw, z, i03, i13, ppad=ppad, nb=25, k=128, fz=256)
    pos_scores = scores[:P]
    neg_scores = scores[P:2 * P]

    proj_s = _mlp(x_s, Pw1, Pb1, Pw2, Pb2)
    proj_d = _mlp(x_d, Pw1, Pb1, Pw2, Pb2)
    return (pos_scores, neg_scores, proj_s, proj_d)


# speedup vs baseline: 1.5166x; 1.0285x over previous
import functools

import jax
import jax.numpy as jnp
from jax import lax
from jax.experimental import pallas as pl
from jax.experimental.pallas import tpu as pltpu
from jax.experimental.pallas import tpu_sc as plsc

N = 10000
E = 160000
M = 2000
P = 50000

NC = 2
NS = 16
NW = NC * NS
L = 16

NP = 10240
MP = 2048
E32 = 163840
EWC = E32 // NW

BM = 1000


def _mesh():
    return plsc.VectorSubcoreMesh(core_axis_name="c", subcore_axis_name="s",
                                  num_cores=NC, num_subcores=NS)


def _seg_sorted(tables, ss, rowptr, *, op, fc, k=64):
    nchunks = len(tables)
    opt = op // NW
    nfc = fc // L

    def body(*refs):
        ss_hbm, rp_hbm = refs[0], refs[1]
        tabs = refs[2:2 + nchunks]
        outs = refs[2 + nchunks:2 + 2 * nchunks]
        srci0, srci1, rowbuf0, rowbuf1, staging, rp_v, sem0, sem1 = (
            refs[2 + 2 * nchunks:])
        cid = lax.axis_index("c")
        sid = lax.axis_index("s")
        w = cid * NS + sid
        lo = w * opt

        pltpu.sync_copy(rp_hbm.at[pl.ds(lo, opt + L)], rp_v)
        b_lo = rp_v[pl.ds(0, L)][0]
        b_hi = rp_v[pl.ds(opt, L)][0]
        e0 = (b_lo // k) * k
        nbt = (b_hi + k - 1) // k - b_lo // k

        zero = jnp.zeros((L,), jnp.float32)

        for c in range(nchunks):
            table = tabs[c]

            def zloop(i, _):
                for t in range(nfc):
                    staging[i, pl.ds(t * L, L)] = zero
                return 0
            lax.fori_loop(0, opt, zloop, 0)

            def compute(base, rowbuf, dcur):
                bend = base + k

                def cond(st):
                    d = st[0]
                    v = rp_v[pl.ds(d, L)]
                    return (d < opt) & (v[0] < bend)

                def wbody(st):
                    d = st[0]
                    v = rp_v[pl.ds(d, L)]
                    es = jnp.maximum(v[0] - base, 0)
                    ee = jnp.minimum(v[1] - base, k)

                    def ebody(e, acc):
                        return tuple(
                            acc[t] + rowbuf[e, pl.ds(t * L, L)]
                            for t in range(nfc))
                    acc = lax.fori_loop(
                        es, ee, ebody, tuple(zero for _ in range(nfc)))
                    for t in range(nfc):
                        staging[d, pl.ds(t * L, L)] = (
                            staging[d, pl.ds(t * L, L)] + acc[t])
                    return (d + 1,)

                d_fin = lax.while_loop(cond, wbody, (dcur,))[0]
                vf = rp_v[pl.ds(d_fin, L)]
                return jnp.where((d_fin > 0) & (vf[0] > bend), d_fin - 1, d_fin)

            @pl.when(nbt > 0)
            def _():
                pltpu.sync_copy(ss_hbm.at[pl.ds(e0, k)], srci0)
                pltpu.async_copy(table.at[srci0], rowbuf0, sem0)

            def bpair(i, dcur):
                b0 = 2 * i
                b1 = b0 + 1
                base0 = e0 + b0 * k
                pltpu.make_async_copy(table.at[srci0], rowbuf0, sem0).wait()

                @pl.when(b1 < nbt)
                def _():
                    pltpu.sync_copy(ss_hbm.at[pl.ds(base0 + k, k)], srci1)
                    pltpu.async_copy(table.at[srci1], rowbuf1, sem1)

                dc1 = compute(base0, rowbuf0, dcur)

                def with_b1(dc):
                    pltpu.make_async_copy(table.at[srci1], rowbuf1, sem1).wait()

                    @pl.when(b1 + 1 < nbt)
                    def _():
                        pltpu.sync_copy(
                            ss_hbm.at[pl.ds(base0 + 2 * k, k)], srci0)
                        pltpu.async_copy(table.at[srci0], rowbuf0, sem0)

                    return compute(base0 + k, rowbuf1, dc)

                return lax.cond(b1 < nbt, with_b1, lambda dc: dc, dc1)

            lax.fori_loop(0, (nbt + 1) // 2, bpair, jnp.int32(0))
            pltpu.sync_copy(staging, outs[c].at[pl.ds(lo, opt)])

    out_type = tuple(jax.ShapeDtypeStruct((op, fc), jnp.float32)
                     for _ in range(nchunks))
    fn = pl.kernel(
        body,
        out_type=out_type,
        mesh=_mesh(),
        compiler_params=pltpu.CompilerParams(needs_layout_passes=False),
        scratch_types=[
            pltpu.VMEM((k,), jnp.int32),
            pltpu.VMEM((k,), jnp.int32),
            pltpu.VMEM((k, fc), jnp.float32),
            pltpu.VMEM((k, fc), jnp.float32),
            pltpu.VMEM((opt, fc), jnp.float32),
            pltpu.VMEM((opt + L,), jnp.int32),
            pltpu.SemaphoreType.DMA,
            pltpu.SemaphoreType.DMA,
        ],
    )
    return fn(ss, rowptr, *tables)


def _counts(dst2, node2, he2):
    def body(dst_hbm, node_hbm, he_hbm, od, on, oh, dv, nv, hv, hd, hn, hh):
        cid = lax.axis_index("c")
        sid = lax.axis_index("s")
        w = cid * NS + sid
        pltpu.sync_copy(dst_hbm.at[pl.ds(w * EWC, EWC)], dv)
        pltpu.sync_copy(node_hbm.at[pl.ds(w * EWC, EWC)], nv)
        pltpu.sync_copy(he_hbm.at[pl.ds(w * EWC, EWC)], hv)

        zero = jnp.zeros((L,), jnp.float32)

        def z1(i, _):
            hd[pl.ds(i * L, L)] = zero
            hn[pl.ds(i * L, L)] = zero
            return 0
        lax.fori_loop(0, NP // L, z1, 0)

        def z2(i, _):
            hh[pl.ds(i * L, L)] = zero
            return 0
        lax.fori_loop(0, MP // L, z2, 0)

        ones = jnp.full((L,), 1.0, jnp.float32)

        def eloop(i, _):
            plsc.addupdate_scatter(hd, [dv[pl.ds(i * L, L)]], ones)
            plsc.addupdate_scatter(hn, [nv[pl.ds(i * L, L)]], ones)
            plsc.addupdate_scatter(hh, [hv[pl.ds(i * L, L)]], ones)
            return 0
        lax.fori_loop(0, EWC // L, eloop, 0)

        pltpu.sync_copy(hd, od.at[pl.ds(w * NP, NP)])
        pltpu.sync_copy(hn, on.at[pl.ds(w * NP, NP)])
        pltpu.sync_copy(hh, oh.at[pl.ds(w * MP, MP)])

    fn = pl.kernel(
        body,
        out_type=(jax.ShapeDtypeStruct((NW * NP,), jnp.float32),
                  jax.ShapeDtypeStruct((NW * NP,), jnp.float32),
                  jax.ShapeDtypeStruct((NW * MP,), jnp.float32)),
        mesh=_mesh(),
        compiler_params=pltpu.CompilerParams(needs_layout_passes=False),
        scratch_types=[
            pltpu.VMEM((EWC,), jnp.int32),
            pltpu.VMEM((EWC,), jnp.int32),
            pltpu.VMEM((EWC,), jnp.int32),
            pltpu.VMEM((NP,), jnp.float32),
            pltpu.VMEM((NP,), jnp.float32),
            pltpu.VMEM((MP,), jnp.float32),
        ],
    )
    cd, cn, ch = fn(dst2, node2, he2)
    return (cd.reshape(NW, NP), cn.reshape(NW, NP), ch.reshape(NW, MP))


def _decoder(zw, z, i03, i13, *, ppad, nb, k, fz):
    pt = ppad // NW

    def body(zw_hbm, z_hbm, i0_hbm, i1_hbm, out, i0r, i1r, r0, r1, sv):
        cid = lax.axis_index("c")
        sid = lax.axis_index("s")
        w = cid * NS + sid

        lane = lax.iota(jnp.int32, L)

        def bloop(b, _):
            base = (w * nb + b) * k
            pltpu.sync_copy(i0_hbm.at[pl.ds(base, k)], i0r)
            pltpu.sync_copy(i1_hbm.at[pl.ds(base, k)], i1r)
            pltpu.sync_copy(zw_hbm.at[i0r], r0)
            pltpu.sync_copy(z_hbm.at[i1r], r1)

            def gloop(g, _):
                vec = jnp.zeros((L,), jnp.float32)
                for j in range(L):
                    p = g * L + j
                    acc = r0[p, pl.ds(0, L)] * r1[p, pl.ds(0, L)]
                    for f in range(1, fz // L):
                        acc = acc + r0[p, pl.ds(f * L, L)] * r1[p, pl.ds(f * L, L)]
                    vec = jnp.where(lane == j, jnp.sum(acc, axis=0), vec)
                sv[pl.ds(b * k + g * L, L)] = vec
                return 0
            lax.fori_loop(0, k // L, gloop, 0)
            return 0
        lax.fori_loop(0, nb, bloop, 0)
        pltpu.sync_copy(sv, out.at[pl.ds(w * pt, pt)])

    fn = pl.kernel(
        body,
        out_type=jax.ShapeDtypeStruct((ppad,), jnp.float32),
        mesh=_mesh(),
        compiler_params=pltpu.CompilerParams(needs_layout_passes=False),
        scratch_types=[
            pltpu.VMEM((k,), jnp.int32),
            pltpu.VMEM((k,), jnp.int32),
            pltpu.VMEM((k, fz), jnp.float32),
            pltpu.VMEM((k, fz), jnp.float32),
            pltpu.VMEM((pt,), jnp.float32),
        ],
    )
    return fn(zw, z, i03, i13)


def _elu(v):
    return jnp.where(v > 0, v, jnp.exp(jnp.minimum(v, 0.0)) - 1.0)


def _inv_body(cd_ref, cn_ref, ch_ref, di_ref, dn_ref, bi_ref):
    d = jnp.sum(cd_ref[...], axis=0) + 1.0
    di_ref[...] = lax.rsqrt(d)[:, None]
    cn = jnp.sum(cn_ref[...], axis=0)
    dn_ref[...] = jnp.where(cn > 0, 1.0 / jnp.where(cn > 0, cn, 1.0), 0.0)[:, None]
    ch = jnp.sum(ch_ref[...], axis=0)
    bi_ref[...] = jnp.where(ch > 0, 1.0 / jnp.where(ch > 0, ch, 1.0), 0.0)[:, None]


def _invs(cd, cn, ch):
    return pl.pallas_call(
        _inv_body,
        out_shape=(jax.ShapeDtypeStruct((NP, 1), jnp.float32),
                   jax.ShapeDtypeStruct((NP, 1), jnp.float32),
                   jax.ShapeDtypeStruct((MP, 1), jnp.float32)),
    )(cd, cn, ch)


def _mm_body(a_ref, w_ref, b_ref, o_ref, *, act):
    acc = jnp.dot(a_ref[...], w_ref[...], preferred_element_type=jnp.float32)
    acc = acc + b_ref[...]
    if act == "elu":
        acc = _elu(acc)
    o_ref[...] = acc


def _mm(a, w, bias=None, act=None, bm=BM):
    m, kk = a.shape
    f = w.shape[1]
    if bias is None:
        bias = jnp.zeros((f,), jnp.float32)
    return pl.pallas_call(
        functools.partial(_mm_body, act=act),
        grid=(m // bm,),
        in_specs=[
            pl.BlockSpec((bm, kk), lambda i: (i, 0)),
            pl.BlockSpec((kk, f), lambda i: (0, 0)),
            pl.BlockSpec((f,), lambda i: (0,)),
        ],
        out_specs=pl.BlockSpec((bm, f), lambda i: (i, 0)),
        out_shape=jax.ShapeDtypeStruct((m, f), jnp.float32),
    )(a, w, bias)


def _mm_scale_body(a_ref, w_ref, s_ref, o_ref):
    o_ref[0] = jnp.dot(a_ref[...], w_ref[...],
                       preferred_element_type=jnp.float32) * s_ref[...]


def _mm_scale_chunks(a, w, scale, bm=BM):
    m, kk = a.shape
    f = w.shape[1]
    c = f // 256
    return pl.pallas_call(
        _mm_scale_body,
        grid=(c, m // bm),
        in_specs=[
            pl.BlockSpec((bm, kk), lambda ci, i: (i, 0)),
            pl.BlockSpec((kk, 256), lambda ci, i: (0, ci)),
            pl.BlockSpec((bm, 1), lambda ci, i: (i, 0)),
        ],
        out_specs=pl.BlockSpec((1, bm, 256), lambda ci, i: (ci, i, 0)),
        out_shape=jax.ShapeDtypeStruct((c, m, 256), jnp.float32),
    )(a, w, scale)


def _gcn_mid_body(p0, p1, hn_ref, s_ref, b_ref, w_ref, o_ref):
    t = jnp.concatenate(
        [p0[...] + hn_ref[0], p1[...] + hn_ref[1]], axis=-1)
    a = _elu(t * s_ref[...] + b_ref[...])
    o_ref[...] = jnp.dot(a, w_ref[...],
                         preferred_element_type=jnp.float32) * s_ref[...]


def _gcn_mid(parts, hn, dinv, b1, w2, bm=BM):
    pspec = pl.BlockSpec((bm, 256), lambda i: (i, 0))
    return pl.pallas_call(
        _gcn_mid_body,
        grid=(N // bm,),
        in_specs=[pspec, pspec,
                  pl.BlockSpec((2, bm, 256), lambda i: (0, i, 0)),
                  pl.BlockSpec((bm, 1), lambda i: (i, 0)),
                  pl.BlockSpec((512,), lambda i: (0,)),
                  pl.BlockSpec((512, 256), lambda i: (0, 0))],
        out_specs=pl.BlockSpec((bm, 256), lambda i: (i, 0)),
        out_shape=jax.ShapeDtypeStruct((N, 256), jnp.float32),
    )(*parts, hn, dinv, b1, w2)


def _comb_body(*refs, has_hn):
    o_ref = refs[-1]
    b_ref = refs[-2]
    s_ref = refs[-3]
    t = refs[0][...]
    if has_hn:
        t = t + refs[1][...]
    o_ref[...] = t * s_ref[...] + b_ref[...]


def _combine(part, hn, scale, bias, bm=BM):
    pspec = pl.BlockSpec((bm, 256), lambda i: (i, 0))
    in_specs = [pspec]
    args = [part]
    if hn is not None:
        in_specs.append(pspec)
        args.append(hn)
    in_specs += [pl.BlockSpec((bm, 1), lambda i: (i, 0)),
                 pl.BlockSpec((256,), lambda i: (0,))]
    args += [scale, bias]
    return pl.pallas_call(
        functools.partial(_comb_body, has_hn=hn is not None),
        grid=(N // bm,),
        in_specs=in_specs,
        out_specs=pl.BlockSpec((bm, 256), lambda i: (i, 0)),
        out_shape=jax.ShapeDtypeStruct((N, 256), jnp.float32),
    )(*args)


def _eagg_body(*refs, nq):
    s_ref = refs[nq]
    o_ref = refs[-1]
    for qi in range(nq):
        o_ref[qi] = refs[qi][:M] * s_ref[...][:M]


def _eagg(qs, binv):
    nq = len(qs)
    return pl.pallas_call(
        functools.partial(_eagg_body, nq=nq),
        in_specs=[pl.BlockSpec((MP, 256), lambda: (0, 0))] * nq
        + [pl.BlockSpec((MP, 1), lambda: (0, 0))],
        out_specs=pl.BlockSpec((nq, M, 256), lambda: (0, 0, 0)),
        out_shape=jax.ShapeDtypeStruct((nq, M, 256), jnp.float32),
        grid=(),
    )(*qs, binv)


def _hyp_mid_body(r0, r1, s_ref, b_ref, w_ref, o_ref):
    t = jnp.concatenate([r0[...], r1[...]], axis=-1)
    a = _elu(t * s_ref[...] + b_ref[...])
    o_ref[...] = jnp.dot(a, w_ref[...], preferred_element_type=jnp.float32)


def _hyp_mid(parts, dinv, bh1, wh2, bm=BM):
    pspec = pl.BlockSpec((bm, 256), lambda i: (i, 0))
    return pl.pallas_call(
        _hyp_mid_body,
        grid=(N // bm,),
        in_specs=[pspec, pspec,
                  pl.BlockSpec((bm, 1), lambda i: (i, 0)),
                  pl.BlockSpec((512,), lambda i: (0,)),
                  pl.BlockSpec((512, 256), lambda i: (0, 0))],
        out_specs=pl.BlockSpec((bm, 256), lambda i: (i, 0)),
        out_shape=jax.ShapeDtypeStruct((N, 256), jnp.float32),
    )(*parts, dinv, bh1, wh2)


def _gate_body(xs_ref, xd_ref, g_ref, w_ref, z_ref, zw_ref):
    a = 1.0 / (1.0 + jnp.exp(-g_ref[0, 0]))
    z = a * xs_ref[...] + (1.0 - a) * xd_ref[...]
    z_ref[...] = z
    zw_ref[...] = jnp.dot(z, w_ref[...], preferred_element_type=jnp.float32)


def _gate_fuse(xs, xd, gate, wdec, bm=BM):
    return pl.pallas_call(
        _gate_body,
        grid=(N // bm,),
        in_specs=[pl.BlockSpec((bm, 256), lambda i: (i, 0)),
                  pl.BlockSpec((bm, 256), lambda i: (i, 0)),
                  pl.BlockSpec((1, 1), lambda i: (0, 0)),
                  pl.BlockSpec((256, 256), lambda i: (0, 0))],
        out_specs=(pl.BlockSpec((bm, 256), lambda i: (i, 0)),
                   pl.BlockSpec((bm, 256), lambda i: (i, 0))),
        out_shape=(jax.ShapeDtypeStruct((N, 256), jnp.float32),
                   jax.ShapeDtypeStruct((N, 256), jnp.float32)),
    )(xs, xd, gate, wdec)


def _mlp_body(a_ref, w1_ref, b1_ref, w2_ref, b2_ref, o_ref):
    t = _elu(jnp.dot(a_ref[...], w1_ref[...],
                     preferred_element_type=jnp.float32) + b1_ref[...])
    o_ref[...] = jnp.dot(t, w2_ref[...],
                         preferred_element_type=jnp.float32) + b2_ref[...]


def _mlp(a, w1, b1, w2, b2, bm=BM):
    return pl.pallas_call(
        _mlp_body,
        grid=(N // bm,),
        in_specs=[pl.BlockSpec((bm, 256), lambda i: (i, 0)),
                  pl.BlockSpec((256, 256), lambda i: (0, 0)),
                  pl.BlockSpec((256,), lambda i: (0,)),
                  pl.BlockSpec((256, 256), lambda i: (0, 0)),
                  pl.BlockSpec((256,), lambda i: (0,))],
        out_specs=pl.BlockSpec((bm, 256), lambda i: (i, 0)),
        out_shape=jax.ShapeDtypeStruct((N, 256), jnp.float32),
    )(a, w1, b1, w2, b2)


def _pad_idx(a, pad_val, shape):
    total = 1
    for s in shape:
        total *= s
    a = a.astype(jnp.int32)
    return jnp.concatenate(
        [a, jnp.full((total - a.shape[0],), pad_val, jnp.int32)]).reshape(shape)


def _sort_edges(s_arr, d_arr, pad_d, op):
    sp = _pad_idx(s_arr, 0, (E32,))
    dp = _pad_idx(d_arr, pad_d, (E32,))
    order = jnp.argsort(dp)
    ss = sp[order]
    dd = dp[order]
    ticks = jnp.arange(op + 1, dtype=jnp.int32)
    rowptr = jnp.searchsorted(dd, ticks, side="left").astype(jnp.int32)
    rowptr = jnp.concatenate(
        [rowptr, jnp.full((15,), E32, jnp.int32)])
    return ss, dd, rowptr


def kernel(x, edge_index, hyperedge_index, pos_edges, neg_edges, W1, b1, W2, b2,
           Wh1, bh1, Wh2, bh2, gate, Wdec, Pw1, Pb1, Pw2, Pb2):
    src, dst = edge_index[0], edge_index[1]
    node, he = hyperedge_index[0], hyperedge_index[1]

    ssg, ddg, rpg = _sort_edges(src, dst, NP - 1, NP)
    ss1, dd1, rp1 = _sort_edges(node, he, MP - 1, MP)
    ss2, dd2, rp2 = _sort_edges(he, node, NP - 1, NP)

    ppad = 102400
    i03 = _pad_idx(jnp.concatenate([pos_edges[0], neg_edges[0]]), 0, (ppad,))
    i13 = _pad_idx(jnp.concatenate([pos_edges[1], neg_edges[1]]), 0, (ppad,))

    cd, cn, ch = _counts(ddg, dd2, dd1)
    dinv_f, ninv_f, binv = _invs(cd, cn, ch)
    dinv = dinv_f[:N]
    ninv = ninv_f[:N]

    hn1 = _mm_scale_chunks(x, W1, dinv)
    p1 = _seg_sorted([hn1[0], hn1[1]], ssg, rpg, op=NP, fc=256)
    hn2 = _gcn_mid(p1, hn1, dinv, b1, W2)
    p2 = _seg_sorted([hn2], ssg, rpg, op=NP, fc=256)[0]
    x_s = _combine(p2, hn2, dinv, b2)

    ones_n = jnp.ones((N, 1), jnp.float32)
    g1 = _mm_scale_chunks(x, Wh1, ones_n)
    q1 = _seg_sorted([g1[0], g1[1]], ss1, rp1, op=MP, fc=256, k=128)
    e1 = _eagg(q1, binv)
    r1 = _seg_sorted([e1[0], e1[1]], ss2, rp2, op=NP, fc=256)
    g2 = _hyp_mid(r1, ninv, bh1, Wh2)
    q2 = _seg_sorted([g2], ss1, rp1, op=MP, fc=256, k=128)
    e2 = _eagg(q2, binv)
    r2 = _seg_sorted([e2[0]], ss2, rp2, op=NP, fc=256)[0]
    x_d = _combine(r2, None, ninv, bh2)

    z, zw = _gate_fuse(x_s, x_d, gate.reshape(1, 1), Wdec)
    scores = _decoder(zw, z, i03, i13, ppad=ppad, nb=25, k=128, fz=256)
    pos_scores = scores[:P]
    neg_scores = scores[P:2 * P]

    proj_s = _mlp(x_s, Pw1, Pb1, Pw2, Pb2)
    proj_d = _mlp(x_d, Pw1, Pb1, Pw2, Pb2)
    return (pos_scores, neg_scores, proj_s, proj_d)
